# baseline (speedup 1.0000x reference)
import functools

import jax
import jax.numpy as jnp
from jax import lax
from jax.experimental import pallas as pl
from jax.experimental.pallas import tpu as pltpu
from jax.experimental.pallas import tpu_sc as plsc


def kernel(x, pos_emb):
    batch = x.shape[0]
    max_len, d_model = pos_emb.shape

    info = plsc.get_sparse_core_info()
    num_workers = info.num_cores * info.num_subcores
    rows_per_worker = max_len // num_workers
    chunk = 64
    n_chunks = rows_per_worker // chunk

    mesh = plsc.VectorSubcoreMesh(core_axis_name="c", subcore_axis_name="s")

    @functools.partial(
        pl.kernel,
        mesh=mesh,
        out_type=jax.ShapeDtypeStruct((batch, max_len, d_model), pos_emb.dtype),
        scratch_types=[
            pltpu.VMEM((chunk, d_model), pos_emb.dtype),
            pltpu.SemaphoreType.DMA,
        ],
    )
    def sc_copy(table_hbm, out_hbm, buf, sem):
        wid = lax.axis_index("s") * info.num_cores + lax.axis_index("c")
        base = wid * rows_per_worker

        def body(i, carry):
            r = base + i * chunk
            pltpu.sync_copy(table_hbm.at[pl.ds(r, chunk)], buf)
            copies = [
                pltpu.make_async_copy(
                    buf, out_hbm.at[b, pl.ds(r, chunk)], sem
                )
                for b in range(batch)
            ]
            for c in copies:
                c.start()
            for c in copies:
                c.wait()
            return carry

        lax.fori_loop(0, n_chunks, body, 0)

    return sc_copy(pos_emb)
